# baseline (speedup 1.0000x reference)
import math

import numpy as np
import jax
import jax.numpy as jnp
from jax import lax
from jax.experimental import pallas as pl
from jax.experimental.pallas import tpu as pltpu
from jax.experimental.pallas import tpu_sc as plsc

_L = 16
_T = 2 ** 14
_F = 2
_N = 524288
_NCORES = 2
_NSUB = 16
_NHALF = _N // _NCORES
_C = 8192
_NCHUNK = _NHALF // _C
_VPC = _C // 16

_b = math.exp((math.log(512) - math.log(16)) / (_L - 1))
_RES = [int(16 * _b ** i) for i in range(_L)]
_SCALES = np.array([np.float32(n / 6.0) for n in _RES], dtype=np.float32)

_P1D = int(np.uint32(2654435761 % (1 << 32)).view(np.int32))
_P2D = int(np.int32(805459861))
_MSK = _T - 1


def _tile_body(xt_hbm, yt_hbm, zt_hbm, ht_hbm, sc_hbm, out_hbm,
               svec_ref, tbl_ref, xin0, xin1, yin0, yin1, zin0, zin1,
               o00, o01, o10, o11,
               sem_in0, sem_in1, sem_out0, sem_out1):
    c = lax.axis_index("c")
    s = lax.axis_index("s")
    li = s
    base0 = c * jnp.int32(_NHALF)
    xin = (xin0, xin1)
    yin = (yin0, yin1)
    zin = (zin0, zin1)
    o0 = (o00, o01)
    o1 = (o10, o11)
    sem_in = (sem_in0, sem_in1)
    sem_out = (sem_out0, sem_out1)

    pltpu.sync_copy(sc_hbm, svec_ref)
    pltpu.sync_copy(ht_hbm.at[li], tbl_ref)
    scale = plsc.load_gather(svec_ref, [jnp.full((16,), li, jnp.int32)])

    def in_copies(k, b):
        base = base0 + k * jnp.int32(_C)
        sl = pl.ds(base, _C)
        return (pltpu.make_async_copy(xt_hbm.at[sl], xin[b], sem_in[b]),
                pltpu.make_async_copy(yt_hbm.at[sl], yin[b], sem_in[b]),
                pltpu.make_async_copy(zt_hbm.at[sl], zin[b], sem_in[b]))

    def out_copies(k, b):
        base = base0 + k * jnp.int32(_C)
        sl = pl.ds(base, _C)
        return (pltpu.make_async_copy(o0[b], out_hbm.at[0, li, sl], sem_out[b]),
                pltpu.make_async_copy(o1[b], out_hbm.at[1, li, sl], sem_out[b]))

    def compute(k, b):
        xb, yb, zb = xin[b], yin[b], zin[b]
        ob0, ob1 = o0[b], o1[b]

        @plsc.parallel_loop(0, _VPC, unroll=8)
        def _vreg(j):
            off = j * jnp.int32(16)

            def coords(v):
                t = (v + 3.0) * scale
                m = t.astype(jnp.int32)
                p = t - m.astype(jnp.float32)
                return m, p

            xm, xp = coords(xb[pl.ds(off, 16)])
            ym, yp = coords(yb[pl.ds(off, 16)])
            zm, zp = coords(zb[pl.ds(off, 16)])

            xm2 = xm & _MSK
            xM2 = (xm + 1) & _MSK
            tym = (ym * _P1D) & _MSK
            tyM = (tym + _P1D) & _MSK
            tzm = (zm * _P2D) & _MSK
            tzM = (tzm + _P2D) & _MSK
            a00 = xm2 ^ tym
            a01 = xm2 ^ tyM
            a10 = xM2 ^ tym
            a11 = xM2 ^ tyM
            i000 = a00 ^ tzm
            i001 = a00 ^ tzM
            i010 = a01 ^ tzm
            i011 = a01 ^ tzM
            i100 = a10 ^ tzm
            i101 = a10 ^ tzM
            i110 = a11 ^ tzm
            i111 = a11 ^ tzM

            xq = 1.0 - xp
            yq = 1.0 - yp
            zq = 1.0 - zp
            w00 = xq * yq
            w01 = xq * yp
            w10 = xp * yq
            w11 = xp * yp
            w000 = w00 * zq
            w001 = w00 * zp
            w010 = w01 * zq
            w011 = w01 * zp
            w100 = w10 * zq
            w101 = w10 * zp
            w110 = w11 * zq
            w111 = w11 * zp

            g = plsc.load_gather
            bc = lambda u: plsc.bitcast(u, jnp.bfloat16)
            pk = lambda w: plsc.pack(w, w, format=plsc.PackFormat.INTERLEAVED)
            t0 = bc(g(tbl_ref, [i000])) * pk(w000)
            t1 = bc(g(tbl_ref, [i001])) * pk(w001)
            t2 = bc(g(tbl_ref, [i010])) * pk(w010)
            t3 = bc(g(tbl_ref, [i011])) * pk(w011)
            t4 = bc(g(tbl_ref, [i100])) * pk(w100)
            t5 = bc(g(tbl_ref, [i101])) * pk(w101)
            t6 = bc(g(tbl_ref, [i110])) * pk(w110)
            t7 = bc(g(tbl_ref, [i111])) * pk(w111)
            acc = ((t0 + t1) + (t2 + t3)) + ((t4 + t5) + (t6 + t7))
            acc0, acc1 = plsc.unpack(acc, format=plsc.PackFormat.INTERLEAVED)
            acc0 = acc0.astype(jnp.float32) if acc0.dtype != jnp.float32 else acc0
            acc1 = acc1.astype(jnp.float32) if acc1.dtype != jnp.float32 else acc1
            ob0[pl.ds(off, 16)] = acc0
            ob1[pl.ds(off, 16)] = acc1

    for cp in in_copies(jnp.int32(0), 0):
        cp.start()
    for cp in in_copies(jnp.int32(1), 1):
        cp.start()

    @pl.loop(0, _NCHUNK // 2)
    def _pipe(kk):
        for b in (0, 1):
            k = kk * jnp.int32(2) + jnp.int32(b)
            for cp in in_copies(k, b):
                cp.wait()

            @pl.when(kk >= 1)
            def _():
                for cp in out_copies(k - 2, b):
                    cp.wait()

            compute(k, b)
            for cp in out_copies(k, b):
                cp.start()

            @pl.when(kk < _NCHUNK // 2 - 1)
            def _():
                for cp in in_copies(k + 2, b):
                    cp.start()

    for b in (0, 1):
        k = jnp.int32(_NCHUNK - 2 + b)
        for cp in out_copies(k, b):
            cp.wait()


@jax.jit
def _encode_sc(xt, yt, zt, ht_flat, scales):
    with jax.numpy_dtype_promotion("standard"), jax.enable_x64(False):
        return _encode_sc_impl(xt, yt, zt, ht_flat, scales)


def _encode_sc_impl(xt, yt, zt, ht_flat, scales):
    mesh = plsc.VectorSubcoreMesh(core_axis_name="c", subcore_axis_name="s")
    f = pl.kernel(
        _tile_body,
        out_type=jax.ShapeDtypeStruct((_F, _L, _N), jnp.float32),
        mesh=mesh,
        scratch_types=[
            pltpu.VMEM((16,), jnp.float32),
            pltpu.VMEM((_T,), jnp.int32),
            pltpu.VMEM((_C,), jnp.float32),
            pltpu.VMEM((_C,), jnp.float32),
            pltpu.VMEM((_C,), jnp.float32),
            pltpu.VMEM((_C,), jnp.float32),
            pltpu.VMEM((_C,), jnp.float32),
            pltpu.VMEM((_C,), jnp.float32),
            pltpu.VMEM((_C,), jnp.float32),
            pltpu.VMEM((_C,), jnp.float32),
            pltpu.VMEM((_C,), jnp.float32),
            pltpu.VMEM((_C,), jnp.float32),
            pltpu.SemaphoreType.DMA,
            pltpu.SemaphoreType.DMA,
            pltpu.SemaphoreType.DMA,
            pltpu.SemaphoreType.DMA,
        ],
        compiler_params=pltpu.CompilerParams(needs_layout_passes=False),
    )
    return f(xt, yt, zt, ht_flat, scales)


def kernel(inputs, hash_table):
    xt = inputs[:, 0]
    yt = inputs[:, 1]
    zt = inputs[:, 2]
    ht_bf = hash_table.astype(jnp.bfloat16)
    ht_flat = jax.lax.bitcast_convert_type(ht_bf, jnp.int32)
    scales = jnp.asarray(_SCALES)
    out = _encode_sc(xt, yt, zt, ht_flat, scales)
    return out.transpose(2, 1, 0).reshape(_N, _L * _F)


# speedup vs baseline: 1.3748x; 1.3748x over previous
import math

import numpy as np
import jax
import jax.numpy as jnp
from jax import lax
from jax.experimental import pallas as pl
from jax.experimental.pallas import tpu as pltpu
from jax.experimental.pallas import tpu_sc as plsc

_L = 16
_T = 2 ** 14
_F = 2
_N = 524288
_NCORES = 2
_NSUB = 16
_NHALF = _N // _NCORES
_C = 8192
_NCHUNK = _NHALF // _C
_VPC = _C // 16

_b = math.exp((math.log(512) - math.log(16)) / (_L - 1))
_RES = [int(16 * _b ** i) for i in range(_L)]
_SCALES = np.array([np.float32(n / 6.0) for n in _RES], dtype=np.float32)

_P1D = int(np.uint32(2654435761 % (1 << 32)).view(np.int32))
_P2D = int(np.int32(805459861))
_MSK = _T - 1


def _tile_body(xt_hbm, yt_hbm, zt_hbm, ht_hbm, sc_hbm, out_hbm,
               svec_ref, tbl_ref, xin0, xin1, yin0, yin1, zin0, zin1,
               o00, o01, o10, o11,
               sem_in0, sem_in1, sem_out0, sem_out1):
    c = lax.axis_index("c")
    s = lax.axis_index("s")
    li = s
    base0 = c * jnp.int32(_NHALF)
    xin = (xin0, xin1)
    yin = (yin0, yin1)
    zin = (zin0, zin1)
    o0 = (o00, o01)
    o1 = (o10, o11)
    sem_in = (sem_in0, sem_in1)
    sem_out = (sem_out0, sem_out1)

    pltpu.sync_copy(sc_hbm, svec_ref)
    pltpu.sync_copy(ht_hbm.at[li], tbl_ref)
    scale = plsc.load_gather(svec_ref, [jnp.full((16,), li, jnp.int32)])

    def in_copies(k, b):
        base = base0 + k * jnp.int32(_C)
        sl = pl.ds(base, _C)
        return (pltpu.make_async_copy(xt_hbm.at[sl], xin[b], sem_in[b]),
                pltpu.make_async_copy(yt_hbm.at[sl], yin[b], sem_in[b]),
                pltpu.make_async_copy(zt_hbm.at[sl], zin[b], sem_in[b]))

    def out_copies(k, b):
        base = base0 + k * jnp.int32(_C)
        sl = pl.ds(base, _C)
        return (pltpu.make_async_copy(o0[b], out_hbm.at[0, li, sl], sem_out[b]),
                pltpu.make_async_copy(o1[b], out_hbm.at[1, li, sl], sem_out[b]))

    def compute(k, b):
        xb, yb, zb = xin[b], yin[b], zin[b]
        ob0, ob1 = o0[b], o1[b]

        @pl.loop(0, _VPC, unroll=8)
        def _vreg(j):
            off = j * jnp.int32(16)

            def coords(v):
                t = (v + 3.0) * scale
                m = t.astype(jnp.int32)
                p = t - m.astype(jnp.float32)
                return m, p

            xm, xp = coords(xb[pl.ds(off, 16)])
            ym, yp = coords(yb[pl.ds(off, 16)])
            zm, zp = coords(zb[pl.ds(off, 16)])

            xm2 = xm
            xM2 = xm + 1
            tym = (ym * _P1D) & _MSK
            tyM = (tym + _P1D) & _MSK
            tzm = (zm * _P2D) & _MSK
            tzM = (tzm + _P2D) & _MSK
            a00 = xm2 ^ tym
            a01 = xm2 ^ tyM
            a10 = xM2 ^ tym
            a11 = xM2 ^ tyM
            i000 = a00 ^ tzm
            i001 = a00 ^ tzM
            i010 = a01 ^ tzm
            i011 = a01 ^ tzM
            i100 = a10 ^ tzm
            i101 = a10 ^ tzM
            i110 = a11 ^ tzm
            i111 = a11 ^ tzM

            xq = 1.0 - xp
            yq = 1.0 - yp
            zq = 1.0 - zp
            w00 = xq * yq
            w01 = xq * yp
            w10 = xp * yq
            w11 = xp * yp
            w000 = w00 * zq
            w001 = w00 * zp
            w010 = w01 * zq
            w011 = w01 * zp
            w100 = w10 * zq
            w101 = w10 * zp
            w110 = w11 * zq
            w111 = w11 * zp

            g = plsc.load_gather
            bc = lambda u: plsc.bitcast(u, jnp.bfloat16)
            pk = lambda w: plsc.pack(w, w, format=plsc.PackFormat.INTERLEAVED)
            t0 = bc(g(tbl_ref, [i000])) * pk(w000)
            t1 = bc(g(tbl_ref, [i001])) * pk(w001)
            t2 = bc(g(tbl_ref, [i010])) * pk(w010)
            t3 = bc(g(tbl_ref, [i011])) * pk(w011)
            t4 = bc(g(tbl_ref, [i100])) * pk(w100)
            t5 = bc(g(tbl_ref, [i101])) * pk(w101)
            t6 = bc(g(tbl_ref, [i110])) * pk(w110)
            t7 = bc(g(tbl_ref, [i111])) * pk(w111)
            acc = ((t0 + t1) + (t2 + t3)) + ((t4 + t5) + (t6 + t7))
            acc0, acc1 = plsc.unpack(acc, format=plsc.PackFormat.INTERLEAVED)
            acc0 = acc0.astype(jnp.float32) if acc0.dtype != jnp.float32 else acc0
            acc1 = acc1.astype(jnp.float32) if acc1.dtype != jnp.float32 else acc1
            ob0[pl.ds(off, 16)] = acc0
            ob1[pl.ds(off, 16)] = acc1

    for cp in in_copies(jnp.int32(0), 0):
        cp.start()
    for cp in in_copies(jnp.int32(1), 1):
        cp.start()

    @pl.loop(0, _NCHUNK // 2)
    def _pipe(kk):
        for b in (0, 1):
            k = kk * jnp.int32(2) + jnp.int32(b)
            for cp in in_copies(k, b):
                cp.wait()

            @pl.when(kk >= 1)
            def _():
                for cp in out_copies(k - 2, b):
                    cp.wait()

            compute(k, b)
            for cp in out_copies(k, b):
                cp.start()

            @pl.when(kk < _NCHUNK // 2 - 1)
            def _():
                for cp in in_copies(k + 2, b):
                    cp.start()

    for b in (0, 1):
        k = jnp.int32(_NCHUNK - 2 + b)
        for cp in out_copies(k, b):
            cp.wait()


@jax.jit
def _encode_sc(xt, yt, zt, ht_flat, scales):
    with jax.numpy_dtype_promotion("standard"), jax.enable_x64(False):
        return _encode_sc_impl(xt, yt, zt, ht_flat, scales)


def _encode_sc_impl(xt, yt, zt, ht_flat, scales):
    mesh = plsc.VectorSubcoreMesh(core_axis_name="c", subcore_axis_name="s")
    f = pl.kernel(
        _tile_body,
        out_type=jax.ShapeDtypeStruct((_F, _L, _N), jnp.float32),
        mesh=mesh,
        scratch_types=[
            pltpu.VMEM((16,), jnp.float32),
            pltpu.VMEM((_T,), jnp.int32),
            pltpu.VMEM((_C,), jnp.float32),
            pltpu.VMEM((_C,), jnp.float32),
            pltpu.VMEM((_C,), jnp.float32),
            pltpu.VMEM((_C,), jnp.float32),
            pltpu.VMEM((_C,), jnp.float32),
            pltpu.VMEM((_C,), jnp.float32),
            pltpu.VMEM((_C,), jnp.float32),
            pltpu.VMEM((_C,), jnp.float32),
            pltpu.VMEM((_C,), jnp.float32),
            pltpu.VMEM((_C,), jnp.float32),
            pltpu.SemaphoreType.DMA,
            pltpu.SemaphoreType.DMA,
            pltpu.SemaphoreType.DMA,
            pltpu.SemaphoreType.DMA,
        ],
        compiler_params=pltpu.CompilerParams(needs_layout_passes=False),
    )
    return f(xt, yt, zt, ht_flat, scales)


def kernel(inputs, hash_table):
    xt = inputs[:, 0]
    yt = inputs[:, 1]
    zt = inputs[:, 2]
    ht_bf = hash_table.astype(jnp.bfloat16)
    ht_flat = jax.lax.bitcast_convert_type(ht_bf, jnp.int32)
    scales = jnp.asarray(_SCALES)
    out = _encode_sc(xt, yt, zt, ht_flat, scales)
    return out.transpose(2, 1, 0).reshape(_N, _L * _F)
